# baseline (speedup 1.0000x reference)
import functools

import jax
import jax.numpy as jnp
from jax import lax
from jax.experimental import pallas as pl
from jax.experimental.pallas import tpu as pltpu
from jax.experimental.pallas import tpu_sc as plsc

B = 4
N = 4096
D = 128
K = 1024
CHUNK = 512

NW = 32
RPW = K // NW
RB = 16


def _exact_t(x):
    ones = jnp.ones((1, 1), jnp.float32)
    return lax.dot_general(ones, x, (((1,), (1,)), ((), ())),
                           preferred_element_type=jnp.float32,
                           precision=lax.Precision.HIGHEST)


IC = 512
JC = 512
RC = 256
NI = N // IC
NJ = N // JC
KR = K // RC


def _topk_kernel(h_ref, w_ref, b_ref, idx_ref, val_ref, sC3, sT3, rkC3, rkT3,
                 idxf3, valf3):
    h = h_ref[0]
    wt = w_ref[...]
    bb = b_ref[0, 0]
    wgt = lax.dot_general(h, wt, (((1,), (0,)), ((), ())),
                          preferred_element_type=jnp.float32)
    s = jax.nn.sigmoid(wgt + bb)
    for i in range(NI):
        sC3[i] = s[i * IC:(i + 1) * IC]
        sT3[i] = _exact_t(s[i * IC:(i + 1) * IC])

    def rank_body(t, _):
        ic = t // NJ
        jc = lax.rem(t, NJ)
        si = sC3[ic]
        sj = sT3[jc]
        prev = jnp.where(jc == 0, jnp.zeros((IC, 1), jnp.float32),
                         rkC3[ic])

        @pl.when(jc < ic)
        def _():
            cnt = jnp.sum((sj >= si).astype(jnp.float32), axis=1,
                          keepdims=True)
            rkC3[ic] = prev + cnt

        @pl.when(jc > ic)
        def _():
            cnt = jnp.sum((sj > si).astype(jnp.float32), axis=1,
                          keepdims=True)
            rkC3[ic] = prev + cnt

        @pl.when(jc == ic)
        def _():
            li = lax.broadcasted_iota(jnp.int32, (IC, 1), 0) + ic * IC
            lj = lax.broadcasted_iota(jnp.int32, (1, JC), 1) + jc * JC
            tie = (sj == si) & (lj < li)
            cnt = jnp.sum(((sj > si) | tie).astype(jnp.float32), axis=1,
                          keepdims=True)
            rkC3[ic] = prev + cnt
        return 0

    lax.fori_loop(0, NI * NJ, rank_body, 0, unroll=False)
    for j in range(NJ):
        rkT3[j] = _exact_t(rkC3[j])

    def sel_body(t, _):
        rc = t // NJ
        jc = lax.rem(t, NJ)
        rj = rkT3[jc]
        sj = sT3[jc]
        lj = (lax.broadcasted_iota(jnp.int32, (1, JC), 1).astype(jnp.float32)
              + jc.astype(jnp.float32) * JC)
        rcol = (lax.broadcasted_iota(jnp.int32, (RC, 1), 0).astype(jnp.float32)
                + rc.astype(jnp.float32) * RC)
        onehot = (rj == rcol).astype(jnp.float32)
        pidx = jnp.sum(onehot * lj, axis=1, keepdims=True)
        pval = jnp.sum(onehot * sj, axis=1, keepdims=True)
        pi_prev = jnp.where(jc == 0, jnp.zeros((RC, 1), jnp.float32),
                            idxf3[rc])
        pv_prev = jnp.where(jc == 0, jnp.zeros((RC, 1), jnp.float32),
                            valf3[rc])
        idxf3[rc] = pi_prev + pidx
        valf3[rc] = pv_prev + pval
        return 0

    lax.fori_loop(0, KR * NJ, sel_body, 0, unroll=False)
    for i in range(KR):
        idx_ref[0, :, i * RC:(i + 1) * RC] = _exact_t(idxf3[i])
        val_ref[0, :, i * RC:(i + 1) * RC] = _exact_t(valf3[i])


def _topk(h, W, b):
    idx, val = pl.pallas_call(
        _topk_kernel,
        grid=(B,),
        in_specs=[
            pl.BlockSpec((1, N, D), lambda i: (i, 0, 0)),
            pl.BlockSpec((D, 1), lambda i: (0, 0)),
            pl.BlockSpec((1, 1), lambda i: (0, 0)),
        ],
        out_specs=[
            pl.BlockSpec((1, 1, K), lambda i: (i, 0, 0)),
            pl.BlockSpec((1, 1, K), lambda i: (i, 0, 0)),
        ],
        out_shape=[
            jax.ShapeDtypeStruct((B, 1, K), jnp.float32),
            jax.ShapeDtypeStruct((B, 1, K), jnp.float32),
        ],
        scratch_shapes=[
            pltpu.VMEM((NI, IC, 1), jnp.float32),
            pltpu.VMEM((NJ, 1, JC), jnp.float32),
            pltpu.VMEM((NI, IC, 1), jnp.float32),
            pltpu.VMEM((NJ, 1, JC), jnp.float32),
            pltpu.VMEM((KR, RC, 1), jnp.float32),
            pltpu.VMEM((KR, RC, 1), jnp.float32),
        ],
    )(h, W.reshape(1, D).T, b.reshape(1, 1))
    return idx.astype(jnp.int32).reshape(B * K), val.reshape(B * K)


def _col_gather(cols_v, rowbuf, outbuf, row0):
    def chunk_body(cc, carry):
        cvec = cols_v[pl.ds(cc * 16, 16)]
        for r in range(RB):
            rvec = jnp.full((16,), r, jnp.int32)
            outbuf[row0 + r, pl.ds(cc * 16, 16)] = plsc.load_gather(
                rowbuf, [rvec, cvec])
        return carry
    lax.fori_loop(0, K // 16, chunk_body, 0, unroll=False)


def _gather_body(g1f, g2f, hf, idx_hbm, val_hbm, gsec, gsen, newh,
                 cols_v, rowbuf, outbuf, hbuf, nhbuf, vals_v, sem):
    cid = lax.axis_index("c")
    sid = lax.axis_index("s")
    wid = sid * 2 + cid
    base = wid * RPW

    for item in range(B):
        pltpu.sync_copy(idx_hbm.at[pl.ds(item * K, K)], cols_v)
        for tbl, dual in [(g1f, item >= 2)] + ([(g2f, False)] if item < 2 else []):
            for t in range(RPW // RB):
                rows = cols_v[pl.ds(base + t * 16, 16)] + item * N
                pltpu.async_copy(tbl.at[rows], rowbuf, sem).wait()
                _col_gather(cols_v, rowbuf, outbuf, t * RB)
            if tbl is g1f:
                pltpu.sync_copy(outbuf, gsec.at[item, pl.ds(base, RPW)])
                if dual:
                    pltpu.sync_copy(outbuf, gsen.at[item, pl.ds(base, RPW)])
            else:
                pltpu.sync_copy(outbuf, gsen.at[item, pl.ds(base, RPW)])
        pltpu.sync_copy(val_hbm.at[pl.ds(item * K + base, RPW)], vals_v)
        for t in range(RPW // RB):
            rows = cols_v[pl.ds(base + t * 16, 16)] + item * N
            pltpu.async_copy(hf.at[rows], hbuf, sem).wait()
            def h_body(r, carry):
                vb = plsc.load_gather(vals_v, [jnp.full((16,), t * RB, jnp.int32) + r])
                for ch in range(D // 16):
                    nhbuf[t * RB + r, pl.ds(ch * 16, 16)] = (
                        hbuf[r, pl.ds(ch * 16, 16)] * vb)
                return carry
            lax.fori_loop(0, RB, h_body, 0, unroll=False)
        pltpu.sync_copy(nhbuf, newh.at[item, pl.ds(base, RPW)])


def _gather(g1, g2, h, idx, val):
    g1f = g1.reshape(B * N, N)
    g2f = g2.reshape(B * N, N)
    hf = h.reshape(B * N, D)
    idx = idx.reshape(B * K)
    val = val.reshape(B * K)
    mesh = plsc.VectorSubcoreMesh(core_axis_name="c", subcore_axis_name="s")
    fn = pl.kernel(
        _gather_body,
        mesh=mesh,
        compiler_params=pltpu.CompilerParams(
            needs_layout_passes=False, use_tc_tiling_on_sc=False),
        out_type=[
            jax.ShapeDtypeStruct((B, K, K), jnp.float32),
            jax.ShapeDtypeStruct((B, K, K), jnp.float32),
            jax.ShapeDtypeStruct((B, K, D), jnp.float32),
        ],
        scratch_types=[
            pltpu.VMEM((K,), jnp.int32),
            pltpu.VMEM((RB, N), jnp.float32),
            pltpu.VMEM((RPW, K), jnp.float32),
            pltpu.VMEM((RB, D), jnp.float32),
            pltpu.VMEM((RPW, D), jnp.float32),
            pltpu.VMEM((RPW,), jnp.float32),
            pltpu.SemaphoreType.DMA,
        ],
    )
    return fn(g1f, g2f, hf, idx, val)


def kernel(g1, g2, h, W, b):
    idx, val = _topk(h, W, b)
    gsec, gsen, newh = _gather(g1, g2, h, idx, val)
    return gsec, gsen, newh


# speedup vs baseline: 1.0574x; 1.0574x over previous
import functools

import jax
import jax.numpy as jnp
from jax import lax
from jax.experimental import pallas as pl
from jax.experimental.pallas import tpu as pltpu
from jax.experimental.pallas import tpu_sc as plsc

B = 4
N = 4096
D = 128
K = 1024
CHUNK = 512

NW = 32
RPW = K // NW
RB = 8


def _exact_t(x):
    ones = jnp.ones((1, 1), jnp.float32)
    return lax.dot_general(ones, x, (((1,), (1,)), ((), ())),
                           preferred_element_type=jnp.float32,
                           precision=lax.Precision.HIGHEST)


IC = 512
JC = 512
RC = 256
NI = N // IC
NJ = N // JC
KR = K // RC


def _topk_kernel(h_ref, w_ref, b_ref, idx_ref, val_ref, sC3, sT3, rkC3, rkT3,
                 idxf3, valf3):
    h = h_ref[0]
    wt = w_ref[...]
    bb = b_ref[0, 0]
    wgt = lax.dot_general(h, wt, (((1,), (0,)), ((), ())),
                          preferred_element_type=jnp.float32)
    s = jax.nn.sigmoid(wgt + bb)
    for i in range(NI):
        sC3[i] = s[i * IC:(i + 1) * IC]
        sT3[i] = _exact_t(s[i * IC:(i + 1) * IC])

    def rank_body(t, _):
        ic = t // NJ
        jc = lax.rem(t, NJ)
        si = sC3[ic]
        sj = sT3[jc]
        prev = jnp.where(jc == 0, jnp.zeros((IC, 1), jnp.float32),
                         rkC3[ic])

        @pl.when(jc < ic)
        def _():
            cnt = jnp.sum((sj >= si).astype(jnp.float32), axis=1,
                          keepdims=True)
            rkC3[ic] = prev + cnt

        @pl.when(jc > ic)
        def _():
            cnt = jnp.sum((sj > si).astype(jnp.float32), axis=1,
                          keepdims=True)
            rkC3[ic] = prev + cnt

        @pl.when(jc == ic)
        def _():
            li = lax.broadcasted_iota(jnp.int32, (IC, 1), 0) + ic * IC
            lj = lax.broadcasted_iota(jnp.int32, (1, JC), 1) + jc * JC
            tie = (sj == si) & (lj < li)
            cnt = jnp.sum(((sj > si) | tie).astype(jnp.float32), axis=1,
                          keepdims=True)
            rkC3[ic] = prev + cnt
        return 0

    lax.fori_loop(0, NI * NJ, rank_body, 0, unroll=False)
    for j in range(NJ):
        rkT3[j] = _exact_t(rkC3[j])

    def sel_body(t, _):
        rc = t // NJ
        jc = lax.rem(t, NJ)
        rj = rkT3[jc]
        sj = sT3[jc]
        lj = (lax.broadcasted_iota(jnp.int32, (1, JC), 1).astype(jnp.float32)
              + jc.astype(jnp.float32) * JC)
        rcol = (lax.broadcasted_iota(jnp.int32, (RC, 1), 0).astype(jnp.float32)
                + rc.astype(jnp.float32) * RC)
        onehot = (rj == rcol).astype(jnp.float32)
        pidx = jnp.sum(onehot * lj, axis=1, keepdims=True)
        pval = jnp.sum(onehot * sj, axis=1, keepdims=True)
        pi_prev = jnp.where(jc == 0, jnp.zeros((RC, 1), jnp.float32),
                            idxf3[rc])
        pv_prev = jnp.where(jc == 0, jnp.zeros((RC, 1), jnp.float32),
                            valf3[rc])
        idxf3[rc] = pi_prev + pidx
        valf3[rc] = pv_prev + pval
        return 0

    lax.fori_loop(0, KR * NJ, sel_body, 0, unroll=False)
    for i in range(KR):
        idx_ref[0, :, i * RC:(i + 1) * RC] = _exact_t(idxf3[i])
        val_ref[0, :, i * RC:(i + 1) * RC] = _exact_t(valf3[i])


def _topk(h, W, b):
    idx, val = pl.pallas_call(
        _topk_kernel,
        grid=(B,),
        in_specs=[
            pl.BlockSpec((1, N, D), lambda i: (i, 0, 0)),
            pl.BlockSpec((D, 1), lambda i: (0, 0)),
            pl.BlockSpec((1, 1), lambda i: (0, 0)),
        ],
        out_specs=[
            pl.BlockSpec((1, 1, K), lambda i: (i, 0, 0)),
            pl.BlockSpec((1, 1, K), lambda i: (i, 0, 0)),
        ],
        out_shape=[
            jax.ShapeDtypeStruct((B, 1, K), jnp.float32),
            jax.ShapeDtypeStruct((B, 1, K), jnp.float32),
        ],
        scratch_shapes=[
            pltpu.VMEM((NI, IC, 1), jnp.float32),
            pltpu.VMEM((NJ, 1, JC), jnp.float32),
            pltpu.VMEM((NI, IC, 1), jnp.float32),
            pltpu.VMEM((NJ, 1, JC), jnp.float32),
            pltpu.VMEM((KR, RC, 1), jnp.float32),
            pltpu.VMEM((KR, RC, 1), jnp.float32),
        ],
    )(h, W.reshape(1, D).T, b.reshape(1, 1))
    return idx.astype(jnp.int32).reshape(B * K), val.reshape(B * K)


def _col_gather(cols_v, rowbuf, outbuf, row0):
    def chunk_body(cc, carry):
        cvec = cols_v[pl.ds(cc * 16, 16)]
        for r in range(RB):
            rvec = jnp.full((16,), r, jnp.int32)
            outbuf[row0 + r, pl.ds(cc * 16, 16)] = plsc.load_gather(
                rowbuf, [rvec, cvec])
        return carry
    lax.fori_loop(0, K // 16, chunk_body, 0, unroll=False)


def _gather_body(g1f, g2f, hf, idx_hbm, val_hbm, gsec, gsen, newh,
                 cols_v, rows_v, rowbuf, rowbuf2, hbuf, nhbuf, vals_v,
                 outbuf, sem, sem2):
    cid = lax.axis_index("c")
    sid = lax.axis_index("s")
    wid = sid * 2 + cid
    base = wid * RPW

    for item in range(B):
        pltpu.sync_copy(idx_hbm.at[pl.ds(item * K, K)], cols_v)
        for t in range(2):
            rows_v[pl.ds(t * 16, 16)] = (
                cols_v[pl.ds(base + t * 16, 16)] + item * N)
        tasks = [(g1f, 0), (g1f, 1), (g1f, 2), (g1f, 3)]
        if item < 2:
            tasks += [(g2f, 0), (g2f, 1), (g2f, 2), (g2f, 3)]
        bufs = (rowbuf, rowbuf2)
        sems = (sem, sem2)
        cps = [pltpu.async_copy(tasks[0][0].at[rows_v.at[pl.ds(0, RB)]],
                                bufs[0], sems[0])]
        for k, (tbl, t) in enumerate(tasks):
            cps[k].wait()
            if k + 1 < len(tasks):
                ntbl, nt = tasks[k + 1]
                cps.append(pltpu.async_copy(
                    ntbl.at[rows_v.at[pl.ds(nt * RB, RB)]],
                    bufs[(k + 1) % 2], sems[(k + 1) % 2]))
            _col_gather(cols_v, bufs[k % 2], outbuf, t * RB)
            if t == 3:
                if tbl is g1f:
                    pltpu.sync_copy(outbuf, gsec.at[item, pl.ds(base, RPW)])
                    if item >= 2:
                        pltpu.sync_copy(outbuf,
                                        gsen.at[item, pl.ds(base, RPW)])
                else:
                    pltpu.sync_copy(outbuf, gsen.at[item, pl.ds(base, RPW)])
        pltpu.sync_copy(val_hbm.at[pl.ds(item * K + base, RPW)], vals_v)
        for t in range(2):
            rows = cols_v[pl.ds(base + t * 16, 16)] + item * N
            pltpu.async_copy(hf.at[rows], hbuf, sem).wait()
            def h_body(r, carry):
                vb = plsc.load_gather(vals_v, [jnp.full((16,), t * 16, jnp.int32) + r])
                for ch in range(D // 16):
                    nhbuf[t * 16 + r, pl.ds(ch * 16, 16)] = (
                        hbuf[r, pl.ds(ch * 16, 16)] * vb)
                return carry
            lax.fori_loop(0, 16, h_body, 0, unroll=False)
        pltpu.sync_copy(nhbuf, newh.at[item, pl.ds(base, RPW)])


def _gather(g1, g2, h, idx, val):
    g1f = g1.reshape(B * N, N)
    g2f = g2.reshape(B * N, N)
    hf = h.reshape(B * N, D)
    idx = idx.reshape(B * K)
    val = val.reshape(B * K)
    mesh = plsc.VectorSubcoreMesh(core_axis_name="c", subcore_axis_name="s")
    fn = pl.kernel(
        _gather_body,
        mesh=mesh,
        compiler_params=pltpu.CompilerParams(
            needs_layout_passes=False, use_tc_tiling_on_sc=False),
        out_type=[
            jax.ShapeDtypeStruct((B, K, K), jnp.float32),
            jax.ShapeDtypeStruct((B, K, K), jnp.float32),
            jax.ShapeDtypeStruct((B, K, D), jnp.float32),
        ],
        scratch_types=[
            pltpu.VMEM((K,), jnp.int32),
            pltpu.VMEM((RPW,), jnp.int32),
            pltpu.VMEM((RB, N), jnp.float32),
            pltpu.VMEM((RB, N), jnp.float32),
            pltpu.VMEM((16, D), jnp.float32),
            pltpu.VMEM((RPW, D), jnp.float32),
            pltpu.VMEM((RPW,), jnp.float32),
            pltpu.VMEM((RPW, K), jnp.float32),
            pltpu.SemaphoreType.DMA,
            pltpu.SemaphoreType.DMA,
        ],
    )
    return fn(g1f, g2f, hf, idx, val)


def kernel(g1, g2, h, W, b):
    idx, val = _topk(h, W, b)
    gsec, gsen, newh = _gather(g1, g2, h, idx, val)
    return gsec, gsen, newh
